# baseline (speedup 1.0000x reference)
import functools

import jax
import jax.numpy as jnp
from jax import lax
from jax.experimental import pallas as pl
from jax.experimental.pallas import tpu as pltpu
from jax.experimental.pallas import tpu_sc as plsc

NC = 2
NS = 16
NW = NC * NS
LANES = 16
CB = 2

_S = (0.9999997070270158, -0.16666577215354136, 0.0083325581176538,
      -0.00019812575520288488, 2.7040512127216925e-06, -2.0534244526315134e-08)
_C = (0.9999994437076939, -0.4999955822859331, 0.041661033519067386,
      -0.001386274996090388, 2.42532298889456e-05, -2.2194129825286184e-07)

_RSQRT_MAGIC = 0x5F3759DF


def _sin_cos(x):
    x2 = x * x
    s = jnp.float32(_S[5])
    for c in _S[4::-1]:
        s = s * x2 + jnp.float32(c)
    s = s * x
    co = jnp.float32(_C[5])
    for c in _C[4::-1]:
        co = co * x2 + jnp.float32(c)
    return s, co


def _sqrt_nr(s):
    i = lax.bitcast_convert_type(s, jnp.int32)
    y = lax.bitcast_convert_type(jnp.int32(_RSQRT_MAGIC) - (i >> 1),
                                 jnp.float32)
    y = y * (jnp.float32(1.5) - jnp.float32(0.5) * s * y * y)
    return s * y


def _dist(a_re, a_im, b_re, b_im):
    d_re = a_re - b_re
    d_im = a_im - b_im
    s = d_re * d_re + d_im * d_im + jnp.float32(1e-12)
    return _sqrt_nr(s)


def _concat_sc_body(re_hbm, im_hbm, out_hbm, vre, vim, cb, *, e, d):
    wid = lax.axis_index("s") * NC + lax.axis_index("c")
    rows = 128
    nfull = e // rows

    def do_chunk(sl, nr):
        pltpu.sync_copy(re_hbm.at[sl], vre.at[pl.ds(0, nr)] if nr != rows
                        else vre)
        pltpu.sync_copy(im_hbm.at[sl], vim.at[pl.ds(0, nr)] if nr != rows
                        else vim)

        def arow(r, _):
            for q in range(d // LANES):
                sq = pl.ds(q * LANES, LANES)
                cb[r, pl.ds(q * LANES, LANES)] = vre[r, sq]
                cb[r, pl.ds(d + q * LANES, LANES)] = vim[r, sq]
            return 0

        lax.fori_loop(0, nr, arow, 0, unroll=2)
        pltpu.sync_copy(cb.at[pl.ds(0, nr)] if nr != rows else cb,
                        out_hbm.at[sl])

    def loop(i, _):
        c = wid + i * NW

        @pl.when(c < nfull)
        def _():
            do_chunk(pl.ds(c * rows, rows), rows)
        return 0

    lax.fori_loop(0, (nfull + NW - 1) // NW, loop, 0)
    rem = e - nfull * rows
    if rem:
        @pl.when(wid == 0)
        def _():
            do_chunk(pl.ds(nfull * rows, rem), rem)


def _concat_sc(entity_re, entity_im):
    e, d = entity_re.shape
    mesh = plsc.VectorSubcoreMesh(core_axis_name="c", subcore_axis_name="s",
                                  num_cores=NC, num_subcores=NS)
    fn = pl.kernel(
        functools.partial(_concat_sc_body, e=e, d=d),
        out_type=jax.ShapeDtypeStruct((e, 2 * d), jnp.float32),
        mesh=mesh,
        compiler_params=pltpu.CompilerParams(needs_layout_passes=False),
        scratch_types=[
            pltpu.VMEM((128, d), jnp.float32),
            pltpu.VMEM((128, d), jnp.float32),
            pltpu.VMEM((128, 2 * d), jnp.float32),
        ],
    )
    return fn(entity_re, entity_im)


def _body(comb, ph_tab, hidx_h, tidx_h, ridx_h, nidx_h,
          pos_out, neg_out,
          hidx_v, tidx_v, ridx_v, nidx_v,
          h_v, t_v, ph_v, negbuf_v, pos_v, neg_v,
          up_sem, nsem0, nsem1,
          *, bpw, k_neg, nch, d_dim):
    wid = lax.axis_index("s") * NC + lax.axis_index("c")
    base = wid * bpw
    ndg = d_dim // LANES
    ng = k_neg // LANES
    lane = lax.iota(jnp.int32, LANES)
    zero = jnp.zeros((LANES,), jnp.float32)

    pltpu.sync_copy(hidx_h.at[pl.ds(base, bpw)], hidx_v)
    pltpu.sync_copy(tidx_h.at[pl.ds(base, bpw)], tidx_v)
    pltpu.sync_copy(ridx_h.at[pl.ds(base, bpw)], ridx_v)
    pltpu.sync_copy(nidx_h.at[wid], nidx_v)

    nsems = (nsem0, nsem1)

    def issue(g, buf):
        idx_sl = nidx_v.at[pl.ds(g * CB * k_neg, CB * k_neg)]
        pltpu.async_copy(comb.at[idx_sl], negbuf_v.at[buf], nsems[buf])

    def drain(g, buf):
        idx_sl = nidx_v.at[pl.ds(g * CB * k_neg, CB * k_neg)]
        pltpu.make_async_copy(comb.at[idx_sl], negbuf_v.at[buf],
                              nsems[buf]).wait()

    issue(0, 0)
    if nch > 1:
        issue(1, 1)

    up = [
        pltpu.async_copy(comb.at[hidx_v], h_v, up_sem),
        pltpu.async_copy(comb.at[tidx_v], t_v, up_sem),
        pltpu.async_copy(ph_tab.at[ridx_v], ph_v, up_sem),
    ]
    for dsc in up:
        dsc.wait()

    def rot_body(bt, _):
        for dg in range(ndg):
            sl_re = pl.ds(dg * LANES, LANES)
            sl_im = pl.ds(d_dim + dg * LANES, LANES)
            s, co = _sin_cos(ph_v[bt, sl_re])
            h_re = h_v[bt, sl_re]
            h_im = h_v[bt, sl_im]
            h_v[bt, sl_re] = h_re * co - h_im * s
            h_v[bt, sl_im] = h_re * s + h_im * co
        return 0

    lax.fori_loop(0, bpw, rot_body, 0)

    dmask = d_dim - 1

    def pos_body(bg, _):
        rows = bg * LANES + lane

        def d_body(dd, acc):
            colr = (lane + dd) & dmask
            coli = colr | d_dim
            a_re = plsc.load_gather(h_v, [rows, colr])
            a_im = plsc.load_gather(h_v, [rows, coli])
            b_re = plsc.load_gather(t_v, [rows, colr])
            b_im = plsc.load_gather(t_v, [rows, coli])
            return acc + _dist(a_re, a_im, b_re, b_im)

        acc = lax.fori_loop(0, d_dim, d_body, zero, unroll=8)
        pos_v[pl.ds(bg * LANES, LANES)] = -acc
        return 0

    lax.fori_loop(0, bpw // LANES, pos_body, 0)

    def chunk_body(i, _):
        for j in range(min(2, nch)):
            g = i * 2 + j
            drain(g, j)
            nb = negbuf_v.at[j]
            for b_loc in range(CB):
                bt = g * CB + b_loc
                bts = lane * 0 + bt
                rowv = [b_loc * k_neg + kg * LANES + lane for kg in range(ng)]

                def t_body(t, accs, bts=bts, rowv=rowv, nb=nb):
                    colr = (lane + t) & dmask
                    coli = colr | d_dim
                    a_re = plsc.load_gather(h_v, [bts, colr])
                    a_im = plsc.load_gather(h_v, [bts, coli])
                    out = []
                    for kg in range(ng):
                        n_re = plsc.load_gather(nb, [rowv[kg], colr])
                        n_im = plsc.load_gather(nb, [rowv[kg], coli])
                        out.append(accs[kg] + _dist(a_re, a_im, n_re, n_im))
                    return tuple(out)

                accs = lax.fori_loop(0, d_dim, t_body, (zero,) * ng,
                                     unroll=4)
                for kg in range(ng):
                    neg_v[bt, pl.ds(kg * LANES, LANES)] = -accs[kg]
            if nch > 2:
                @pl.when(g < nch - 2)
                def _():
                    issue(g + 2, j)
        return 0

    lax.fori_loop(0, (nch + 1) // 2, chunk_body, 0)

    pltpu.sync_copy(pos_v, pos_out.at[pl.ds(base, bpw)])
    pltpu.sync_copy(neg_v, neg_out.at[pl.ds(base, bpw)])


def kernel(head_idx, relation_idx, tail_idx, negative_idx,
           entity_re, entity_im, rel_phase):
    b, k_neg = negative_idx.shape
    d = entity_re.shape[1]
    assert d == 64 and b % NW == 0
    bpw = b // NW
    assert bpw % CB == 0
    nch = bpw // CB

    comb = _concat_sc(entity_re, entity_im)
    ph_pad = jnp.concatenate([rel_phase, rel_phase], axis=1)
    nidx = negative_idx.astype(jnp.int32).reshape(NW, bpw * k_neg)

    mesh = plsc.VectorSubcoreMesh(core_axis_name="c", subcore_axis_name="s",
                                  num_cores=NC, num_subcores=NS)
    fn = pl.kernel(
        functools.partial(_body, bpw=bpw, k_neg=k_neg, nch=nch, d_dim=d),
        out_type=(jax.ShapeDtypeStruct((b,), jnp.float32),
                  jax.ShapeDtypeStruct((b, d), jnp.float32)),
        mesh=mesh,
        compiler_params=pltpu.CompilerParams(needs_layout_passes=False,
                                             use_tc_tiling_on_sc=False),
        scratch_types=[
            pltpu.VMEM((bpw,), jnp.int32),
            pltpu.VMEM((bpw,), jnp.int32),
            pltpu.VMEM((bpw,), jnp.int32),
            pltpu.VMEM((bpw * k_neg,), jnp.int32),
            pltpu.VMEM((bpw, 2 * d), jnp.float32),
            pltpu.VMEM((bpw, 2 * d), jnp.float32),
            pltpu.VMEM((bpw, 2 * d), jnp.float32),
            pltpu.VMEM((2, CB * k_neg, 2 * d), jnp.float32),
            pltpu.VMEM((bpw,), jnp.float32),
            pltpu.VMEM((bpw, d), jnp.float32),
            pltpu.SemaphoreType.DMA,
            pltpu.SemaphoreType.DMA,
            pltpu.SemaphoreType.DMA,
        ],
    )
    return fn(comb, ph_pad,
              head_idx.astype(jnp.int32), tail_idx.astype(jnp.int32),
              relation_idx.astype(jnp.int32), nidx)


# speedup vs baseline: 2.1391x; 2.1391x over previous
import functools

import jax
import jax.numpy as jnp
from jax import lax
from jax.experimental import pallas as pl
from jax.experimental.pallas import tpu as pltpu
from jax.experimental.pallas import tpu_sc as plsc

NC = 2
NS = 16
NW = NC * NS
LANES = 16
CB = 2

_S = (0.9999997070270158, -0.16666577215354136, 0.0083325581176538,
      -0.00019812575520288488, 2.7040512127216925e-06, -2.0534244526315134e-08)
_C = (0.9999994437076939, -0.4999955822859331, 0.041661033519067386,
      -0.001386274996090388, 2.42532298889456e-05, -2.2194129825286184e-07)

_RSQRT_MAGIC = 0x5F3759DF


def _sin_cos(x):
    x2 = x * x
    s = jnp.float32(_S[5])
    for c in _S[4::-1]:
        s = s * x2 + jnp.float32(c)
    s = s * x
    co = jnp.float32(_C[5])
    for c in _C[4::-1]:
        co = co * x2 + jnp.float32(c)
    return s, co


def _sqrt_nr(s):
    i = lax.bitcast_convert_type(s, jnp.int32)
    y = lax.bitcast_convert_type(jnp.int32(_RSQRT_MAGIC) - (i >> 1),
                                 jnp.float32)
    y = y * (jnp.float32(1.5) - jnp.float32(0.5) * s * y * y)
    return s * y


def _dist(a_re, a_im, b_re, b_im):
    d_re = a_re - b_re
    d_im = a_im - b_im
    s = d_re * d_re + d_im * d_im + jnp.float32(1e-12)
    return _sqrt_nr(s)


def _body(comb, ph_tab, hidx_h, tidx_h, ridx_h, nidx_h,
          pos_out, neg_out,
          hidx_v, tidx_v, ridx_v, nidx_v,
          h_v, t_v, ph_v, negbuf_v, pos_v, neg_v,
          up_sem, nsem0, nsem1,
          *, bpw, k_neg, nch, d_dim):
    wid = lax.axis_index("s") * NC + lax.axis_index("c")
    base = wid * bpw
    ndg = d_dim // LANES
    ng = k_neg // LANES
    lane = lax.iota(jnp.int32, LANES)
    zero = jnp.zeros((LANES,), jnp.float32)

    pltpu.sync_copy(hidx_h.at[pl.ds(base, bpw)], hidx_v)
    pltpu.sync_copy(tidx_h.at[pl.ds(base, bpw)], tidx_v)
    pltpu.sync_copy(ridx_h.at[pl.ds(base, bpw)], ridx_v)
    pltpu.sync_copy(nidx_h.at[wid], nidx_v)

    nsems = (nsem0, nsem1)

    def issue(g, buf):
        idx_sl = nidx_v.at[pl.ds(g * CB * k_neg, CB * k_neg)]
        pltpu.async_copy(comb.at[idx_sl], negbuf_v.at[buf], nsems[buf])

    def drain(g, buf):
        idx_sl = nidx_v.at[pl.ds(g * CB * k_neg, CB * k_neg)]
        pltpu.make_async_copy(comb.at[idx_sl], negbuf_v.at[buf],
                              nsems[buf]).wait()

    issue(0, 0)
    if nch > 1:
        issue(1, 1)

    up = [
        pltpu.async_copy(comb.at[hidx_v], h_v, up_sem),
        pltpu.async_copy(comb.at[tidx_v], t_v, up_sem),
        pltpu.async_copy(ph_tab.at[ridx_v], ph_v, up_sem),
    ]
    for dsc in up:
        dsc.wait()

    def rot_body(bt, _):
        for dg in range(ndg):
            sl_re = pl.ds(dg * LANES, LANES)
            sl_im = pl.ds(d_dim + dg * LANES, LANES)
            s, co = _sin_cos(ph_v[bt, sl_re])
            h_re = h_v[bt, sl_re]
            h_im = h_v[bt, sl_im]
            h_v[bt, sl_re] = h_re * co - h_im * s
            h_v[bt, sl_im] = h_re * s + h_im * co
        return 0

    lax.fori_loop(0, bpw, rot_body, 0)

    dmask = d_dim - 1

    def pos_body(bg, _):
        rows = bg * LANES + lane

        def d_body(dd, acc):
            colr = (lane + dd) & dmask
            coli = colr | d_dim
            a_re = plsc.load_gather(h_v, [rows, colr])
            a_im = plsc.load_gather(h_v, [rows, coli])
            b_re = plsc.load_gather(t_v, [rows, colr])
            b_im = plsc.load_gather(t_v, [rows, coli])
            return acc + _dist(a_re, a_im, b_re, b_im)

        acc = lax.fori_loop(0, d_dim, d_body, zero, unroll=8)
        pos_v[pl.ds(bg * LANES, LANES)] = -acc
        return 0

    lax.fori_loop(0, bpw // LANES, pos_body, 0)

    def chunk_body(i, _):
        for j in range(min(2, nch)):
            g = i * 2 + j
            drain(g, j)
            nb = negbuf_v.at[j]
            for b_loc in range(CB):
                bt = g * CB + b_loc
                bts = lane * 0 + bt
                rowv = [b_loc * k_neg + kg * LANES + lane for kg in range(ng)]

                def t_body(t, accs, bts=bts, rowv=rowv, nb=nb):
                    colr = (lane + t) & dmask
                    coli = colr | d_dim
                    a_re = plsc.load_gather(h_v, [bts, colr])
                    a_im = plsc.load_gather(h_v, [bts, coli])
                    out = []
                    for kg in range(ng):
                        n_re = plsc.load_gather(nb, [rowv[kg], colr])
                        n_im = plsc.load_gather(nb, [rowv[kg], coli])
                        out.append(accs[kg] + _dist(a_re, a_im, n_re, n_im))
                    return tuple(out)

                accs = lax.fori_loop(0, d_dim, t_body, (zero,) * ng,
                                     unroll=8)
                for kg in range(ng):
                    neg_v[bt, pl.ds(kg * LANES, LANES)] = -accs[kg]
            if nch > 2:
                @pl.when(g < nch - 2)
                def _():
                    issue(g + 2, j)
        return 0

    lax.fori_loop(0, (nch + 1) // 2, chunk_body, 0)

    pltpu.sync_copy(pos_v, pos_out.at[pl.ds(base, bpw)])
    pltpu.sync_copy(neg_v, neg_out.at[pl.ds(base, bpw)])


def kernel(head_idx, relation_idx, tail_idx, negative_idx,
           entity_re, entity_im, rel_phase):
    b, k_neg = negative_idx.shape
    d = entity_re.shape[1]
    assert d == 64 and b % NW == 0
    bpw = b // NW
    assert bpw % CB == 0
    nch = bpw // CB

    comb = jnp.concatenate([entity_re, entity_im], axis=1)
    ph_pad = jnp.concatenate([rel_phase, rel_phase], axis=1)
    nidx = negative_idx.astype(jnp.int32).reshape(NW, bpw * k_neg)

    mesh = plsc.VectorSubcoreMesh(core_axis_name="c", subcore_axis_name="s",
                                  num_cores=NC, num_subcores=NS)
    fn = pl.kernel(
        functools.partial(_body, bpw=bpw, k_neg=k_neg, nch=nch, d_dim=d),
        out_type=(jax.ShapeDtypeStruct((b,), jnp.float32),
                  jax.ShapeDtypeStruct((b, d), jnp.float32)),
        mesh=mesh,
        compiler_params=pltpu.CompilerParams(needs_layout_passes=False,
                                             use_tc_tiling_on_sc=False),
        scratch_types=[
            pltpu.VMEM((bpw,), jnp.int32),
            pltpu.VMEM((bpw,), jnp.int32),
            pltpu.VMEM((bpw,), jnp.int32),
            pltpu.VMEM((bpw * k_neg,), jnp.int32),
            pltpu.VMEM((bpw, 2 * d), jnp.float32),
            pltpu.VMEM((bpw, 2 * d), jnp.float32),
            pltpu.VMEM((bpw, 2 * d), jnp.float32),
            pltpu.VMEM((2, CB * k_neg, 2 * d), jnp.float32),
            pltpu.VMEM((bpw,), jnp.float32),
            pltpu.VMEM((bpw, d), jnp.float32),
            pltpu.SemaphoreType.DMA,
            pltpu.SemaphoreType.DMA,
            pltpu.SemaphoreType.DMA,
        ],
    )
    return fn(comb, ph_pad,
              head_idx.astype(jnp.int32), tail_idx.astype(jnp.int32),
              relation_idx.astype(jnp.int32), nidx)


# speedup vs baseline: 2.4395x; 1.1404x over previous
import functools

import jax
import jax.numpy as jnp
from jax import lax
from jax.experimental import pallas as pl
from jax.experimental.pallas import tpu as pltpu
from jax.experimental.pallas import tpu_sc as plsc

NC = 2
NS = 16
NW = NC * NS
LANES = 16
CB = 2

_S = (0.9999997070270158, -0.16666577215354136, 0.0083325581176538,
      -0.00019812575520288488, 2.7040512127216925e-06, -2.0534244526315134e-08)
_C = (0.9999994437076939, -0.4999955822859331, 0.041661033519067386,
      -0.001386274996090388, 2.42532298889456e-05, -2.2194129825286184e-07)

_RSQRT_MAGIC = 0x5F3759DF


def _sin_cos(x):
    x2 = x * x
    s = jnp.float32(_S[5])
    for c in _S[4::-1]:
        s = s * x2 + jnp.float32(c)
    s = s * x
    co = jnp.float32(_C[5])
    for c in _C[4::-1]:
        co = co * x2 + jnp.float32(c)
    return s, co


def _sqrt_nr(s):
    i = lax.bitcast_convert_type(s, jnp.int32)
    y = lax.bitcast_convert_type(jnp.int32(_RSQRT_MAGIC) - (i >> 1),
                                 jnp.float32)
    y = y * (jnp.float32(1.5) - jnp.float32(0.5) * s * y * y)
    return s * y


def _dist(a_re, a_im, b_re, b_im):
    d_re = a_re - b_re
    d_im = a_im - b_im
    s = d_re * d_re + d_im * d_im + jnp.float32(1e-12)
    return _sqrt_nr(s)


def _body(comb, ph_tab, hidx_h, tidx_h, ridx_h, nidx_h,
          pos_out, neg_out,
          hidx_v, tidx_v, ridx_v, nidx_v,
          h_v, t_v, ph_v, negbuf_v, pos_v, neg_v,
          up_sem, nsem0, nsem1,
          *, bpw, k_neg, nch, d_dim):
    wid = lax.axis_index("s") * NC + lax.axis_index("c")
    base = wid * bpw
    ndg = d_dim // LANES
    ng = k_neg // LANES
    lane = lax.iota(jnp.int32, LANES)
    zero = jnp.zeros((LANES,), jnp.float32)

    pltpu.sync_copy(hidx_h.at[pl.ds(base, bpw)], hidx_v)
    pltpu.sync_copy(tidx_h.at[pl.ds(base, bpw)], tidx_v)
    pltpu.sync_copy(ridx_h.at[pl.ds(base, bpw)], ridx_v)
    pltpu.sync_copy(nidx_h.at[wid], nidx_v)

    nsems = (nsem0, nsem1)

    def issue(g, buf):
        idx_sl = nidx_v.at[pl.ds(g * CB * k_neg, CB * k_neg)]
        pltpu.async_copy(comb.at[idx_sl], negbuf_v.at[buf], nsems[buf])

    def drain(g, buf):
        idx_sl = nidx_v.at[pl.ds(g * CB * k_neg, CB * k_neg)]
        pltpu.make_async_copy(comb.at[idx_sl], negbuf_v.at[buf],
                              nsems[buf]).wait()

    issue(0, 0)
    if nch > 1:
        issue(1, 1)

    up = [
        pltpu.async_copy(comb.at[hidx_v], h_v, up_sem),
        pltpu.async_copy(comb.at[tidx_v], t_v, up_sem),
        pltpu.async_copy(ph_tab.at[ridx_v], ph_v, up_sem),
    ]
    for dsc in up:
        dsc.wait()

    def rot_body(bt, _):
        for dg in range(ndg):
            sl_re = pl.ds(dg * LANES, LANES)
            sl_im = pl.ds(d_dim + dg * LANES, LANES)
            s, co = _sin_cos(ph_v[bt, sl_re])
            h_re = h_v[bt, sl_re]
            h_im = h_v[bt, sl_im]
            h_v[bt, sl_re] = h_re * co - h_im * s
            h_v[bt, sl_im] = h_re * s + h_im * co
        return 0

    lax.fori_loop(0, bpw, rot_body, 0)

    dmask = d_dim - 1

    def pos_body(bg, _):
        rows = bg * LANES + lane

        def d_body(dd, acc):
            colr = (lane + dd) & dmask
            coli = colr | d_dim
            a_re = plsc.load_gather(h_v, [rows, colr])
            a_im = plsc.load_gather(h_v, [rows, coli])
            b_re = plsc.load_gather(t_v, [rows, colr])
            b_im = plsc.load_gather(t_v, [rows, coli])
            return acc + _dist(a_re, a_im, b_re, b_im)

        acc = lax.fori_loop(0, d_dim, d_body, zero, unroll=8)
        pos_v[pl.ds(bg * LANES, LANES)] = -acc
        return 0

    lax.fori_loop(0, bpw // LANES, pos_body, 0)

    def chunk_body(i, _):
        for j in range(min(2, nch)):
            g = i * 2 + j
            drain(g, j)
            nb = negbuf_v.at[j]
            for b_loc in range(CB):
                bt = g * CB + b_loc
                bts = lane * 0 + bt
                rowv = [b_loc * k_neg + kg * LANES + lane for kg in range(ng)]

                def t_body(t, accs, bts=bts, rowv=rowv, nb=nb):
                    colr = (lane + t) & dmask
                    coli = colr | d_dim
                    a_re = plsc.load_gather(h_v, [bts, colr])
                    a_im = plsc.load_gather(h_v, [bts, coli])
                    out = []
                    for kg in range(ng):
                        n_re = plsc.load_gather(nb, [rowv[kg], colr])
                        n_im = plsc.load_gather(nb, [rowv[kg], coli])
                        out.append(accs[kg] + _dist(a_re, a_im, n_re, n_im))
                    return tuple(out)

                accs = lax.fori_loop(0, d_dim, t_body, (zero,) * ng,
                                     unroll=4)
                for kg in range(ng):
                    neg_v[bt, pl.ds(kg * LANES, LANES)] = -accs[kg]
            if nch > 2:
                @pl.when(g < nch - 2)
                def _():
                    issue(g + 2, j)
        return 0

    lax.fori_loop(0, (nch + 1) // 2, chunk_body, 0)

    pltpu.sync_copy(pos_v, pos_out.at[pl.ds(base, bpw)])
    pltpu.sync_copy(neg_v, neg_out.at[pl.ds(base, bpw)])


def kernel(head_idx, relation_idx, tail_idx, negative_idx,
           entity_re, entity_im, rel_phase):
    b, k_neg = negative_idx.shape
    d = entity_re.shape[1]
    assert d == 64 and b % NW == 0
    bpw = b // NW
    assert bpw % CB == 0
    nch = bpw // CB

    comb = jnp.concatenate([entity_re, entity_im], axis=1)
    ph_pad = jnp.concatenate([rel_phase, rel_phase], axis=1)
    nidx = negative_idx.astype(jnp.int32).reshape(NW, bpw * k_neg)

    mesh = plsc.VectorSubcoreMesh(core_axis_name="c", subcore_axis_name="s",
                                  num_cores=NC, num_subcores=NS)
    fn = pl.kernel(
        functools.partial(_body, bpw=bpw, k_neg=k_neg, nch=nch, d_dim=d),
        out_type=(jax.ShapeDtypeStruct((b,), jnp.float32),
                  jax.ShapeDtypeStruct((b, d), jnp.float32)),
        mesh=mesh,
        compiler_params=pltpu.CompilerParams(needs_layout_passes=False,
                                             use_tc_tiling_on_sc=False),
        scratch_types=[
            pltpu.VMEM((bpw,), jnp.int32),
            pltpu.VMEM((bpw,), jnp.int32),
            pltpu.VMEM((bpw,), jnp.int32),
            pltpu.VMEM((bpw * k_neg,), jnp.int32),
            pltpu.VMEM((bpw, 2 * d), jnp.float32),
            pltpu.VMEM((bpw, 2 * d), jnp.float32),
            pltpu.VMEM((bpw, 2 * d), jnp.float32),
            pltpu.VMEM((2, CB * k_neg, 2 * d), jnp.float32),
            pltpu.VMEM((bpw,), jnp.float32),
            pltpu.VMEM((bpw, d), jnp.float32),
            pltpu.SemaphoreType.DMA,
            pltpu.SemaphoreType.DMA,
            pltpu.SemaphoreType.DMA,
        ],
    )
    return fn(comb, ph_pad,
              head_idx.astype(jnp.int32), tail_idx.astype(jnp.int32),
              relation_idx.astype(jnp.int32), nidx)


# speedup vs baseline: 3.0630x; 1.2556x over previous
import functools

import jax
import jax.numpy as jnp
from jax import lax
from jax.experimental import pallas as pl
from jax.experimental.pallas import tpu as pltpu
from jax.experimental.pallas import tpu_sc as plsc

NC = 2
NS = 16
NW = NC * NS
LANES = 16
CB = 2

_S = (0.9999997070270158, -0.16666577215354136, 0.0083325581176538,
      -0.00019812575520288488, 2.7040512127216925e-06, -2.0534244526315134e-08)
_C = (0.9999994437076939, -0.4999955822859331, 0.041661033519067386,
      -0.001386274996090388, 2.42532298889456e-05, -2.2194129825286184e-07)

_RSQRT_MAGIC = 0x5F3759DF


def _sin_cos(x):
    x2 = x * x
    s = jnp.float32(_S[5])
    for c in _S[4::-1]:
        s = s * x2 + jnp.float32(c)
    s = s * x
    co = jnp.float32(_C[5])
    for c in _C[4::-1]:
        co = co * x2 + jnp.float32(c)
    return s, co


def _sqrt_nr(s):
    i = lax.bitcast_convert_type(s, jnp.int32)
    y = lax.bitcast_convert_type(jnp.int32(_RSQRT_MAGIC) - (i >> 1),
                                 jnp.float32)
    y = y * (jnp.float32(1.5) - jnp.float32(0.5) * s * y * y)
    return s * y


def _dist(a_re, a_im, b_re, b_im):
    d_re = a_re - b_re
    d_im = a_im - b_im
    s = d_re * d_re + d_im * d_im + jnp.float32(1e-12)
    return _sqrt_nr(s)


def _body(comb, ph_tab, hidx_h, tidx_h, ridx_h, nidx_h,
          pos_out, neg_out,
          hidx_v, tidx_v, ridx_v, nidx_v,
          h_v, t_v, ph_v, negbuf_v, pos_v, neg_v,
          up_sem, nsem0, nsem1,
          *, bpw, k_neg, nch, d_dim):
    wid = lax.axis_index("s") * NC + lax.axis_index("c")
    base = wid * bpw
    ndg = d_dim // LANES
    ng = k_neg // LANES
    lane = lax.iota(jnp.int32, LANES)
    zero = jnp.zeros((LANES,), jnp.float32)

    pltpu.sync_copy(hidx_h.at[pl.ds(base, bpw)], hidx_v)
    pltpu.sync_copy(tidx_h.at[pl.ds(base, bpw)], tidx_v)
    pltpu.sync_copy(ridx_h.at[pl.ds(base, bpw)], ridx_v)
    pltpu.sync_copy(nidx_h.at[wid], nidx_v)

    nsems = (nsem0, nsem1)

    def issue(g, buf):
        idx_sl = nidx_v.at[pl.ds(g * CB * k_neg, CB * k_neg)]
        pltpu.async_copy(comb.at[idx_sl], negbuf_v.at[buf], nsems[buf])

    def drain(g, buf):
        idx_sl = nidx_v.at[pl.ds(g * CB * k_neg, CB * k_neg)]
        pltpu.make_async_copy(comb.at[idx_sl], negbuf_v.at[buf],
                              nsems[buf]).wait()

    issue(0, 0)
    if nch > 1:
        issue(1, 1)

    up = [
        pltpu.async_copy(comb.at[hidx_v], h_v, up_sem),
        pltpu.async_copy(comb.at[tidx_v], t_v, up_sem),
        pltpu.async_copy(ph_tab.at[ridx_v], ph_v, up_sem),
    ]
    for dsc in up:
        dsc.wait()

    def rot_body(bt, _):
        for dg in range(ndg):
            sl_re = pl.ds(dg * LANES, LANES)
            sl_im = pl.ds(d_dim + dg * LANES, LANES)
            s, co = _sin_cos(ph_v[bt, sl_re])
            h_re = h_v[bt, sl_re]
            h_im = h_v[bt, sl_im]
            h_v[bt, sl_re] = h_re * co - h_im * s
            h_v[bt, sl_im] = h_re * s + h_im * co
        return 0

    lax.fori_loop(0, bpw, rot_body, 0)

    dmask = d_dim - 1

    def pos_body(bg, _):
        rows = bg * LANES + lane

        def d_body(dd, acc):
            colr = (lane + dd) & dmask
            coli = colr | d_dim
            a_re = plsc.load_gather(h_v, [rows, colr])
            a_im = plsc.load_gather(h_v, [rows, coli])
            b_re = plsc.load_gather(t_v, [rows, colr])
            b_im = plsc.load_gather(t_v, [rows, coli])
            return acc + _dist(a_re, a_im, b_re, b_im)

        acc = lax.fori_loop(0, d_dim, d_body, zero, unroll=8)
        pos_v[pl.ds(bg * LANES, LANES)] = -acc
        return 0

    lax.fori_loop(0, bpw // LANES, pos_body, 0)

    def chunk_body(i, _):
        for j in range(min(2, nch)):
            g = i * 2 + j
            drain(g, j)
            nb = negbuf_v.at[j]
            for b_loc in range(CB):
                bt = g * CB + b_loc
                bts = lane * 0 + bt
                rowv = [b_loc * k_neg + kg * LANES + lane for kg in range(ng)]

                def t_body(t, accs, bts=bts, rowv=rowv, nb=nb):
                    colr = (lane + t) & dmask
                    coli = colr | d_dim
                    a_re = plsc.load_gather(h_v, [bts, colr])
                    a_im = plsc.load_gather(h_v, [bts, coli])
                    out = []
                    for kg in range(ng):
                        n_re = plsc.load_gather(nb, [rowv[kg], colr])
                        n_im = plsc.load_gather(nb, [rowv[kg], coli])
                        out.append(accs[kg] + _dist(a_re, a_im, n_re, n_im))
                    return tuple(out)

                accs = lax.fori_loop(0, d_dim, t_body, (zero,) * ng,
                                     unroll=4)
                for kg in range(ng):
                    neg_v[bt, pl.ds(kg * LANES, LANES)] = -accs[kg]
            if nch > 2:
                @pl.when(g < nch - 2)
                def _():
                    issue(g + 2, j)
        return 0

    lax.fori_loop(0, (nch + 1) // 2, chunk_body, 0)

    pltpu.sync_copy(pos_v, pos_out.at[pl.ds(base, bpw)])
    pltpu.sync_copy(neg_v, neg_out.at[pl.ds(base, bpw)])


def kernel(head_idx, relation_idx, tail_idx, negative_idx,
           entity_re, entity_im, rel_phase):
    b, k_neg = negative_idx.shape
    d = entity_re.shape[1]
    assert d == 64 and b % NW == 0
    bpw = b // NW
    assert bpw % CB == 0
    nch = bpw // CB

    comb = jnp.stack([entity_re, entity_im],
                     axis=1).reshape(entity_re.shape[0], 2 * d)
    ph_pad = jnp.concatenate([rel_phase, rel_phase], axis=1)
    nidx = negative_idx.astype(jnp.int32).reshape(NW, bpw * k_neg)

    mesh = plsc.VectorSubcoreMesh(core_axis_name="c", subcore_axis_name="s",
                                  num_cores=NC, num_subcores=NS)
    fn = pl.kernel(
        functools.partial(_body, bpw=bpw, k_neg=k_neg, nch=nch, d_dim=d),
        out_type=(jax.ShapeDtypeStruct((b,), jnp.float32),
                  jax.ShapeDtypeStruct((b, d), jnp.float32)),
        mesh=mesh,
        compiler_params=pltpu.CompilerParams(needs_layout_passes=False,
                                             use_tc_tiling_on_sc=False),
        scratch_types=[
            pltpu.VMEM((bpw,), jnp.int32),
            pltpu.VMEM((bpw,), jnp.int32),
            pltpu.VMEM((bpw,), jnp.int32),
            pltpu.VMEM((bpw * k_neg,), jnp.int32),
            pltpu.VMEM((bpw, 2 * d), jnp.float32),
            pltpu.VMEM((bpw, 2 * d), jnp.float32),
            pltpu.VMEM((bpw, 2 * d), jnp.float32),
            pltpu.VMEM((2, CB * k_neg, 2 * d), jnp.float32),
            pltpu.VMEM((bpw,), jnp.float32),
            pltpu.VMEM((bpw, d), jnp.float32),
            pltpu.SemaphoreType.DMA,
            pltpu.SemaphoreType.DMA,
            pltpu.SemaphoreType.DMA,
        ],
    )
    return fn(comb, ph_pad,
              head_idx.astype(jnp.int32), tail_idx.astype(jnp.int32),
              relation_idx.astype(jnp.int32), nidx)
